# Initial kernel scaffold; baseline (speedup 1.0000x reference)
import jax
import jax.numpy as jnp
from jax.experimental import pallas as pl


def kernel(atoms0, residues0, same_neigh0, diff_neigh0, atoms1, residues1, same_neigh1, diff_neigh1, Wv, Wr, Wsr, Wdr):
    raise NotImplementedError("write your pallas kernel here")


# speedup vs baseline: 9.9938x; 9.9938x over previous
import functools

import jax
import jax.numpy as jnp
from jax import lax
from jax.experimental import pallas as pl
from jax.experimental.pallas import tpu as pltpu
from jax.experimental.pallas import tpu_sc as plsc

N = 50000
K = 10
F = 128
NA = 12
NR = 23
LANES = 16

NW = 32
BPW = 1568
NPAD = NW * BPW
C = 8
CK = C * K
NCHUNK = BPW // C
NBUF = 2

_sc_mesh = plsc.VectorSubcoreMesh(core_axis_name="c", subcore_axis_name="s")


@functools.partial(
    pl.kernel,
    mesh=_sc_mesh,
    compiler_params=pltpu.CompilerParams(use_tc_tiling_on_sc=False),
    out_type=[jax.ShapeDtypeStruct((NPAD, LANES), jnp.float32)] * 4,
    scratch_types=[
        pltpu.VMEM((NCHUNK, CK), jnp.int32),
        pltpu.VMEM((CK, LANES), jnp.float32),
        pltpu.VMEM((CK, LANES), jnp.float32),
        pltpu.VMEM((BPW, LANES), jnp.float32),
        pltpu.SemaphoreType.DMA,
        pltpu.SemaphoreType.DMA,
    ],
)
def _sc_mean_gather(t0, i00, i01, t1, i10, i11,
                    o00, o01, o10, o11,
                    idx_v, buf0, buf1, out_v, sem0, sem1):
    wid = lax.axis_index("s") * 2 + lax.axis_index("c")
    bufs = (buf0, buf1)
    sems = (sem0, sem1)

    def do_pair(idx_hbm, table_hbm, out_hbm):
        pltpu.sync_copy(idx_hbm.at[wid], idx_v)
        for b in range(NBUF):
            pltpu.async_copy(table_hbm.at[idx_v.at[b]], bufs[b], sems[b])

        def body(j, _):
            for b in range(NBUF):
                ch = j * NBUF + b
                pltpu.make_async_copy(
                    table_hbm.at[idx_v.at[ch]], bufs[b], sems[b]).wait()
                for i in range(C):
                    s = bufs[b][i * K, :]
                    for k in range(1, K):
                        s = s + bufs[b][i * K + k, :]
                    out_v[ch * C + i, :] = s * (1.0 / K)
                nxt = ch + NBUF

                @pl.when(nxt < NCHUNK)
                def _fire():
                    pltpu.async_copy(
                        table_hbm.at[idx_v.at[nxt]], bufs[b], sems[b])
            return ()

        lax.fori_loop(0, NCHUNK // NBUF, body, ())
        pltpu.sync_copy(out_v, out_hbm.at[pl.ds(wid * BPW, BPW)])

    do_pair(i00, t0, o00)
    do_pair(i01, t0, o01)
    do_pair(i10, t1, o10)
    do_pair(i11, t1, o11)


BT = 2048


def _tc_fused(a_ref, r_ref, s_ref, d_ref, wv_ref, wr_ref, wsr_ref, wdr_ref,
              o_ref):
    acc = jnp.dot(a_ref[...], wv_ref[...], preferred_element_type=jnp.float32)
    acc = acc + jnp.dot(r_ref[...], wr_ref[...],
                        preferred_element_type=jnp.float32)
    acc = acc + jnp.dot(s_ref[...], wsr_ref[...],
                        preferred_element_type=jnp.float32)
    acc = acc + jnp.dot(d_ref[...], wdr_ref[...],
                        preferred_element_type=jnp.float32)
    o_ref[...] = jnp.maximum(acc, 0.0)


_tc_call = pl.pallas_call(
    _tc_fused,
    grid=(pl.cdiv(N, BT),),
    in_specs=[
        pl.BlockSpec((BT, LANES), lambda i: (i, 0)),
        pl.BlockSpec((BT, 24), lambda i: (i, 0)),
        pl.BlockSpec((BT, LANES), lambda i: (i, 0)),
        pl.BlockSpec((BT, LANES), lambda i: (i, 0)),
        pl.BlockSpec((LANES, F), lambda i: (0, 0)),
        pl.BlockSpec((24, F), lambda i: (0, 0)),
        pl.BlockSpec((LANES, F), lambda i: (0, 0)),
        pl.BlockSpec((LANES, F), lambda i: (0, 0)),
    ],
    out_specs=pl.BlockSpec((BT, F), lambda i: (i, 0)),
    out_shape=jax.ShapeDtypeStruct((N, F), jnp.float32),
)


def _prep_idx(neigh):
    flat = neigh.reshape(-1)
    flat = jnp.pad(flat, (0, (NPAD - N) * K))
    return flat.reshape(NW, NCHUNK, CK)


def kernel(atoms0, residues0, same_neigh0, diff_neigh0,
           atoms1, residues1, same_neigh1, diff_neigh1,
           Wv, Wr, Wsr, Wdr):
    a0p = jnp.pad(atoms0, ((0, 0), (0, LANES - NA)))
    a1p = jnp.pad(atoms1, ((0, 0), (0, LANES - NA)))
    r0p = jnp.pad(residues0, ((0, 0), (0, 24 - NR)))
    r1p = jnp.pad(residues1, ((0, 0), (0, 24 - NR)))
    wv = jnp.pad(Wv, ((0, LANES - NA), (0, 0)))
    wr = jnp.pad(Wr, ((0, 24 - NR), (0, 0)))
    wsr = jnp.pad(Wsr, ((0, LANES - NA), (0, 0)))
    wdr = jnp.pad(Wdr, ((0, LANES - NA), (0, 0)))

    agg00, agg01, agg10, agg11 = _sc_mean_gather(
        a0p, _prep_idx(same_neigh0), _prep_idx(diff_neigh0),
        a1p, _prep_idx(same_neigh1), _prep_idx(diff_neigh1))

    out0 = _tc_call(a0p, r0p, agg00, agg01, wv, wr, wsr, wdr)
    out1 = _tc_call(a1p, r1p, agg10, agg11, wv, wr, wsr, wdr)
    return ((out0, same_neigh0, diff_neigh0), (out1, same_neigh1, diff_neigh1))


# speedup vs baseline: 12.5898x; 1.2598x over previous
import functools

import jax
import jax.numpy as jnp
from jax import lax
from jax.experimental import pallas as pl
from jax.experimental.pallas import tpu as pltpu
from jax.experimental.pallas import tpu_sc as plsc

N = 50000
K = 10
F = 128
NA = 12
NR = 23
LANES = 16

NW = 32
BPW = 1568
NPAD = NW * BPW
C = 8
CK = C * K
NCHUNK = BPW // C
NBUF = 7

_sc_mesh = plsc.VectorSubcoreMesh(core_axis_name="c", subcore_axis_name="s")


@functools.partial(
    pl.kernel,
    mesh=_sc_mesh,
    compiler_params=pltpu.CompilerParams(use_tc_tiling_on_sc=False),
    out_type=[jax.ShapeDtypeStruct((NPAD, LANES), jnp.float32)] * 4,
    scratch_types=(
        [pltpu.VMEM((NCHUNK, CK), jnp.int32)]
        + [pltpu.VMEM((CK, LANES), jnp.float32)] * NBUF
        + [pltpu.VMEM((BPW, LANES), jnp.float32)]
        + [pltpu.SemaphoreType.DMA] * NBUF
    ),
)
def _sc_mean_gather(t0, i00, i01, t1, i10, i11,
                    o00, o01, o10, o11,
                    idx_v, *scratch):
    wid = lax.axis_index("s") * 2 + lax.axis_index("c")
    bufs = scratch[:NBUF]
    out_v = scratch[NBUF]
    sems = scratch[NBUF + 1:]

    def do_pair(idx_hbm, table_hbm, out_hbm):
        pltpu.sync_copy(idx_hbm.at[wid], idx_v)
        for b in range(NBUF):
            pltpu.async_copy(table_hbm.at[idx_v.at[b]], bufs[b], sems[b])

        def body(j, _):
            for b in range(NBUF):
                ch = j * NBUF + b
                pltpu.make_async_copy(
                    table_hbm.at[idx_v.at[ch]], bufs[b], sems[b]).wait()
                for i in range(C):
                    s = bufs[b][i * K, :]
                    for k in range(1, K):
                        s = s + bufs[b][i * K + k, :]
                    out_v[ch * C + i, :] = s * (1.0 / K)
                nxt = ch + NBUF

                @pl.when(nxt < NCHUNK)
                def _fire():
                    pltpu.async_copy(
                        table_hbm.at[idx_v.at[nxt]], bufs[b], sems[b])
            return ()

        lax.fori_loop(0, NCHUNK // NBUF, body, ())
        pltpu.sync_copy(out_v, out_hbm.at[pl.ds(wid * BPW, BPW)])

    do_pair(i00, t0, o00)
    do_pair(i01, t0, o01)
    do_pair(i10, t1, o10)
    do_pair(i11, t1, o11)


BT = 2048


def _tc_fused(a_ref, r_ref, s_ref, d_ref, wv_ref, wr_ref, wsr_ref, wdr_ref,
              o_ref):
    acc = jnp.dot(a_ref[...], wv_ref[...], preferred_element_type=jnp.float32)
    acc = acc + jnp.dot(r_ref[...], wr_ref[...],
                        preferred_element_type=jnp.float32)
    acc = acc + jnp.dot(s_ref[...], wsr_ref[...],
                        preferred_element_type=jnp.float32)
    acc = acc + jnp.dot(d_ref[...], wdr_ref[...],
                        preferred_element_type=jnp.float32)
    o_ref[...] = jnp.maximum(acc, 0.0)


_tc_call = pl.pallas_call(
    _tc_fused,
    grid=(pl.cdiv(N, BT),),
    in_specs=[
        pl.BlockSpec((BT, LANES), lambda i: (i, 0)),
        pl.BlockSpec((BT, 24), lambda i: (i, 0)),
        pl.BlockSpec((BT, LANES), lambda i: (i, 0)),
        pl.BlockSpec((BT, LANES), lambda i: (i, 0)),
        pl.BlockSpec((LANES, F), lambda i: (0, 0)),
        pl.BlockSpec((24, F), lambda i: (0, 0)),
        pl.BlockSpec((LANES, F), lambda i: (0, 0)),
        pl.BlockSpec((LANES, F), lambda i: (0, 0)),
    ],
    out_specs=pl.BlockSpec((BT, F), lambda i: (i, 0)),
    out_shape=jax.ShapeDtypeStruct((N, F), jnp.float32),
)


def _prep_idx(neigh):
    flat = neigh.reshape(-1)
    flat = jnp.pad(flat, (0, (NPAD - N) * K))
    return flat.reshape(NW, NCHUNK, CK)


def kernel(atoms0, residues0, same_neigh0, diff_neigh0,
           atoms1, residues1, same_neigh1, diff_neigh1,
           Wv, Wr, Wsr, Wdr):
    a0p = jnp.pad(atoms0, ((0, 0), (0, LANES - NA)))
    a1p = jnp.pad(atoms1, ((0, 0), (0, LANES - NA)))
    r0p = jnp.pad(residues0, ((0, 0), (0, 24 - NR)))
    r1p = jnp.pad(residues1, ((0, 0), (0, 24 - NR)))
    wv = jnp.pad(Wv, ((0, LANES - NA), (0, 0)))
    wr = jnp.pad(Wr, ((0, 24 - NR), (0, 0)))
    wsr = jnp.pad(Wsr, ((0, LANES - NA), (0, 0)))
    wdr = jnp.pad(Wdr, ((0, LANES - NA), (0, 0)))

    agg00, agg01, agg10, agg11 = _sc_mean_gather(
        a0p, _prep_idx(same_neigh0), _prep_idx(diff_neigh0),
        a1p, _prep_idx(same_neigh1), _prep_idx(diff_neigh1))

    out0 = _tc_call(a0p, r0p, agg00, agg01, wv, wr, wsr, wdr)
    out1 = _tc_call(a1p, r1p, agg10, agg11, wv, wr, wsr, wdr)
    return ((out0, same_neigh0, diff_neigh0), (out1, same_neigh1, diff_neigh1))
